# Initial kernel scaffold; baseline (speedup 1.0000x reference)
import jax
import jax.numpy as jnp
from jax.experimental import pallas as pl


def kernel(q, k_sf, k_df1, k_df2, k_all, memory):
    raise NotImplementedError("write your pallas kernel here")


# speedup vs baseline: 1.7125x; 1.7125x over previous
import jax
import jax.numpy as jnp
from jax.experimental import pallas as pl
from jax.experimental.pallas import tpu as pltpu

_B, _D, _Q = 256, 128, 65536
_SCALE = 10.0
_BQ = 2048
_NMEM = _Q // _BQ
_NGRID = _NMEM + 1


def _tc_body(q_ref, ksf_ref, kdf1_ref, kdf2_ref, kall_ref, mem_ref,
             out_ref, newmem_ref, prev_row):
    j = pl.program_id(0)

    @pl.when(j == 0)
    def _():
        newmem_ref[0:_B, :] = kall_ref[...]
        newmem_ref[_B:_BQ, :] = mem_ref[_B:_BQ, :]

    @pl.when(j > 0)
    def _():
        newmem_ref[...] = mem_ref[...]

    m = mem_ref[...]
    m_shift = jnp.concatenate([prev_row[...], m[: _BQ - 1, :]], axis=0)
    prev_row[...] = m[_BQ - 1 : _BQ, :]

    t = jax.lax.dot_general(
        q_ref[...], m_shift,
        dimension_numbers=(((1,), (1,)), ((), ())),
        preferred_element_type=jnp.float32,
    ) * _SCALE
    out_ref[0:_B, :] = t
    out_ref[_B:2 * _B, :] = t
    out_ref[2 * _B:3 * _B, :] = t

    @pl.when(j == 0)
    def _():
        q = q_ref[...]
        p_sf = jnp.sum(q * ksf_ref[...], axis=1, keepdims=True)
        p_df1 = jnp.sum(q * kdf1_ref[...], axis=1, keepdims=True)
        p_df2 = jnp.sum(q * kdf2_ref[...], axis=1, keepdims=True)
        pos = jnp.concatenate([p_sf, p_df1, p_df2], axis=0) * _SCALE
        out_ref[:, 0:1] = pos


def _rep_spec():
    return pl.BlockSpec((_B, _D), lambda j: (0, 0))


def _mem_index(j):
    return (jnp.minimum(j, _NMEM - 1), 0)


def kernel(q, k_sf, k_df1, k_df2, k_all, memory):
    out, new_memory = pl.pallas_call(
        _tc_body,
        grid=(_NGRID,),
        in_specs=[
            _rep_spec(), _rep_spec(), _rep_spec(), _rep_spec(), _rep_spec(),
            pl.BlockSpec((_BQ, _D), _mem_index),
        ],
        out_specs=[
            pl.BlockSpec((3 * _B, _BQ), lambda j: (0, j)),
            pl.BlockSpec((_BQ, _D), _mem_index),
        ],
        out_shape=[
            jax.ShapeDtypeStruct((3 * _B, _Q + 1), jnp.float32),
            jax.ShapeDtypeStruct((_Q, _D), jnp.float32),
        ],
        scratch_shapes=[pltpu.VMEM((1, _D), jnp.float32)],
        compiler_params=pltpu.CompilerParams(
            dimension_semantics=("arbitrary",),
        ),
    )(q, k_sf, k_df1, k_df2, k_all, memory)
    return out, new_memory
